# baseline (speedup 1.0000x reference)
import jax
import jax.numpy as jnp
from jax.experimental import pallas as pl
from jax.experimental.pallas import tpu as pltpu

EPS = 1e-6
CHUNK = 2048


def _kernel(act_ref, cand_ref, inp_ref, enc_ref, ins_ref, mask_ref,
            out_ref, acc_ref):
    b = pl.program_id(0)
    i = pl.program_id(1)
    nc = pl.num_programs(1)

    @pl.when(i == 0)
    def _init():
        acc_ref[...] = jnp.zeros_like(acc_ref)

    a = act_ref[b]
    lane = a % 128
    win = inp_ref[0]
    lane_ids = jax.lax.broadcasted_iota(jnp.int32, win.shape, 1)
    sel = jnp.sum(jnp.where(lane_ids == lane, win, 0.0), axis=1)

    e = enc_ref[0]
    c = e.shape[1]
    logits = jax.lax.dot_general(
        e, sel[:, None], (((1,), (0,)), ((), ())),
        preferred_element_type=jnp.float32)[:, 0]
    logits = logits * (1.0 / jnp.sqrt(jnp.float32(c)))
    pred = jax.nn.sigmoid(logits)

    m = (mask_ref[0, 0] > 0.5).astype(jnp.float32)
    g = (ins_ref[0, 0] > 0.5).astype(jnp.float32)

    p = pred * m
    t = g * m
    pt = p * t + (1.0 - p) * (1.0 - t)
    focal = -((1.0 - pt) ** 2) * jnp.log(pt + EPS) * m

    def lanes(v):
        return jnp.sum(v.reshape(CHUNK // 128, 128), axis=0)

    acc_ref[0, :] += lanes(focal)
    acc_ref[1, :] += lanes(p * t)
    acc_ref[2, :] += lanes(p)
    acc_ref[3, :] += lanes(t)
    acc_ref[4, :] += lanes(m)

    @pl.when(i == nc - 1)
    def _fin():
        focal_sum = jnp.sum(acc_ref[0, :])
        inter = jnp.sum(acc_ref[1, :])
        sum_p = jnp.sum(acc_ref[2, :])
        sum_t = jnp.sum(acc_ref[3, :])
        mask_sum = jnp.sum(acc_ref[4, :])
        focal_loss = focal_sum / (mask_sum + EPS)
        dice_loss = 1.0 - (2.0 * inter + EPS) / (sum_p + sum_t + EPS)
        loss_atten = (0.5 * focal_loss + dice_loss) * sum_t
        out_ref[0, 0, :] = jnp.full((128,), loss_atten / (mask_sum + EPS))


def kernel(input, encode, ins_seg, mask, actions, candidate_idx):
    b, c, h, w = input.shape
    hw = h * w
    n_ins = ins_seg.shape[1]
    nc = hw // CHUNK

    inp_flat = input.reshape(b, c, hw)
    ins_flat = ins_seg.reshape(b * n_ins, 1, hw)
    mask3 = mask.reshape(b, 1, hw)

    grid_spec = pltpu.PrefetchScalarGridSpec(
        num_scalar_prefetch=2,
        grid=(b, nc),
        in_specs=[
            pl.BlockSpec((1, c, 128),
                         lambda bi, ci, act, cand: (bi, 0, act[bi] // 128)),
            pl.BlockSpec((1, CHUNK, c),
                         lambda bi, ci, act, cand: (bi, ci, 0)),
            pl.BlockSpec((1, 1, CHUNK),
                         lambda bi, ci, act, cand:
                         (bi * n_ins + cand[bi], 0, ci)),
            pl.BlockSpec((1, 1, CHUNK),
                         lambda bi, ci, act, cand: (bi, 0, ci)),
        ],
        out_specs=pl.BlockSpec((1, 1, 128),
                               lambda bi, ci, act, cand: (bi, 0, 0)),
        scratch_shapes=[pltpu.VMEM((8, 128), jnp.float32)],
    )

    out = pl.pallas_call(
        _kernel,
        grid_spec=grid_spec,
        out_shape=jax.ShapeDtypeStruct((b, 1, 128), jnp.float32),
    )(actions, candidate_idx, inp_flat, encode, ins_flat, mask3)
    return out[:, 0, 0]


# speedup vs baseline: 1.4202x; 1.4202x over previous
import jax
import jax.numpy as jnp
from jax.experimental import pallas as pl
from jax.experimental.pallas import tpu as pltpu

EPS = 1e-6
CHUNK = 4096
ROWS = CHUNK // 128


def _kernel(act_ref, cand_ref, inp_ref, enc_ref, ins_ref, mask_ref,
            out_ref, acc_ref):
    b = pl.program_id(0)
    i = pl.program_id(1)
    nc = pl.num_programs(1)

    @pl.when(i == 0)
    def _init():
        acc_ref[...] = jnp.zeros_like(acc_ref)

    a = act_ref[b]
    lane = a % 128
    win = inp_ref[0]
    lane_ids = jax.lax.broadcasted_iota(jnp.int32, win.shape, 1)
    sel = jnp.sum(jnp.where(lane_ids == lane, win, 0.0), axis=1)

    e = enc_ref[0]
    c = e.shape[1]
    logits = jax.lax.dot_general(
        sel[None, :], e, (((1,), (1,)), ((), ())),
        preferred_element_type=jnp.float32)
    l2 = logits.reshape(ROWS, 128) * (1.0 / jnp.sqrt(jnp.float32(c)))
    pred = jax.nn.sigmoid(l2)

    m = (mask_ref[0] > 0.5).astype(jnp.float32)
    g = (ins_ref[0] > 0.5).astype(jnp.float32)

    p = pred * m
    t = g * m
    pt = p * t + (1.0 - p) * (1.0 - t)
    one_m_pt = 1.0 - pt
    focal = -(one_m_pt * one_m_pt) * jnp.log(pt + EPS) * m

    acc_ref[0, :] += jnp.sum(focal, axis=0)
    acc_ref[1, :] += jnp.sum(p * t, axis=0)
    acc_ref[2, :] += jnp.sum(p, axis=0)
    acc_ref[3, :] += jnp.sum(t, axis=0)
    acc_ref[4, :] += jnp.sum(m, axis=0)

    @pl.when(i == nc - 1)
    def _fin():
        focal_sum = jnp.sum(acc_ref[0, :])
        inter = jnp.sum(acc_ref[1, :])
        sum_p = jnp.sum(acc_ref[2, :])
        sum_t = jnp.sum(acc_ref[3, :])
        mask_sum = jnp.sum(acc_ref[4, :])
        focal_loss = focal_sum / (mask_sum + EPS)
        dice_loss = 1.0 - (2.0 * inter + EPS) / (sum_p + sum_t + EPS)
        loss_atten = (0.5 * focal_loss + dice_loss) * sum_t
        out_ref[0, 0, :] = jnp.full((128,), loss_atten / (mask_sum + EPS))


def kernel(input, encode, ins_seg, mask, actions, candidate_idx):
    b, c, h, w = input.shape
    hw = h * w
    n_ins = ins_seg.shape[1]
    nc = hw // CHUNK

    inp_flat = input.reshape(b, c, hw)
    ins_rows = ins_seg.reshape(b * n_ins, hw // 128, 128)
    mask_rows = mask.reshape(b, hw // 128, 128)

    grid_spec = pltpu.PrefetchScalarGridSpec(
        num_scalar_prefetch=2,
        grid=(b, nc),
        in_specs=[
            pl.BlockSpec((1, c, 128),
                         lambda bi, ci, act, cand: (bi, 0, act[bi] // 128)),
            pl.BlockSpec((1, CHUNK, c),
                         lambda bi, ci, act, cand: (bi, ci, 0)),
            pl.BlockSpec((1, ROWS, 128),
                         lambda bi, ci, act, cand:
                         (bi * n_ins + cand[bi], ci, 0)),
            pl.BlockSpec((1, ROWS, 128),
                         lambda bi, ci, act, cand: (bi, ci, 0)),
        ],
        out_specs=pl.BlockSpec((1, 1, 128),
                               lambda bi, ci, act, cand: (bi, 0, 0)),
        scratch_shapes=[pltpu.VMEM((8, 128), jnp.float32)],
    )

    out = pl.pallas_call(
        _kernel,
        grid_spec=grid_spec,
        out_shape=jax.ShapeDtypeStruct((b, 1, 128), jnp.float32),
    )(actions, candidate_idx, inp_flat, encode, ins_rows, mask_rows)
    return out[:, 0, 0]


# speedup vs baseline: 1.4720x; 1.0365x over previous
import jax
import jax.numpy as jnp
from jax.experimental import pallas as pl
from jax.experimental.pallas import tpu as pltpu

EPS = 1e-6
CHUNK = 8192
ROWS = CHUNK // 128


def _kernel(act_ref, cand_ref, inp_ref, enc_ref, ins_ref, mask_ref,
            out_ref, acc_ref):
    b = pl.program_id(0)
    i = pl.program_id(1)
    nc = pl.num_programs(1)

    @pl.when(i == 0)
    def _init():
        acc_ref[...] = jnp.zeros_like(acc_ref)

    a = act_ref[b]
    lane = a % 128
    win = inp_ref[0]
    lane_ids = jax.lax.broadcasted_iota(jnp.int32, win.shape, 1)
    sel = jnp.sum(jnp.where(lane_ids == lane, win, 0.0), axis=1)

    e = enc_ref[0]
    c = e.shape[1]
    logits = jax.lax.dot_general(
        sel[None, :], e, (((1,), (1,)), ((), ())),
        preferred_element_type=jnp.float32)
    l2 = logits.reshape(ROWS, 128) * (1.0 / jnp.sqrt(jnp.float32(c)))
    pred = jax.nn.sigmoid(l2)

    m = (mask_ref[0] > 0.5).astype(jnp.float32)
    g = (ins_ref[0] > 0.5).astype(jnp.float32)

    p = pred * m
    t = g * m
    pt = p * t + (1.0 - p) * (1.0 - t)
    one_m_pt = 1.0 - pt
    focal = -(one_m_pt * one_m_pt) * jnp.log(pt + EPS) * m

    acc_ref[0, :] += jnp.sum(focal, axis=0)
    acc_ref[1, :] += jnp.sum(p * t, axis=0)
    acc_ref[2, :] += jnp.sum(p, axis=0)
    acc_ref[3, :] += jnp.sum(t, axis=0)
    acc_ref[4, :] += jnp.sum(m, axis=0)

    @pl.when(i == nc - 1)
    def _fin():
        focal_sum = jnp.sum(acc_ref[0, :])
        inter = jnp.sum(acc_ref[1, :])
        sum_p = jnp.sum(acc_ref[2, :])
        sum_t = jnp.sum(acc_ref[3, :])
        mask_sum = jnp.sum(acc_ref[4, :])
        focal_loss = focal_sum / (mask_sum + EPS)
        dice_loss = 1.0 - (2.0 * inter + EPS) / (sum_p + sum_t + EPS)
        loss_atten = (0.5 * focal_loss + dice_loss) * sum_t
        out_ref[0, 0, :] = jnp.full((128,), loss_atten / (mask_sum + EPS))


def kernel(input, encode, ins_seg, mask, actions, candidate_idx):
    b, c, h, w = input.shape
    hw = h * w
    n_ins = ins_seg.shape[1]
    nc = hw // CHUNK

    inp_flat = input.reshape(b, c, hw)
    ins_rows = ins_seg.reshape(b * n_ins, hw // 128, 128)
    mask_rows = mask.reshape(b, hw // 128, 128)

    grid_spec = pltpu.PrefetchScalarGridSpec(
        num_scalar_prefetch=2,
        grid=(b, nc),
        in_specs=[
            pl.BlockSpec((1, c, 128),
                         lambda bi, ci, act, cand: (bi, 0, act[bi] // 128)),
            pl.BlockSpec((1, CHUNK, c),
                         lambda bi, ci, act, cand: (bi, ci, 0)),
            pl.BlockSpec((1, ROWS, 128),
                         lambda bi, ci, act, cand:
                         (bi * n_ins + cand[bi], ci, 0)),
            pl.BlockSpec((1, ROWS, 128),
                         lambda bi, ci, act, cand: (bi, ci, 0)),
        ],
        out_specs=pl.BlockSpec((1, 1, 128),
                               lambda bi, ci, act, cand: (bi, 0, 0)),
        scratch_shapes=[pltpu.VMEM((8, 128), jnp.float32)],
    )

    out = pl.pallas_call(
        _kernel,
        grid_spec=grid_spec,
        out_shape=jax.ShapeDtypeStruct((b, 1, 128), jnp.float32),
    )(actions, candidate_idx, inp_flat, encode, ins_rows, mask_rows)
    return out[:, 0, 0]
